# baseline (speedup 1.0000x reference)
import functools

import jax
import jax.numpy as jnp
from jax import lax
from jax.experimental import pallas as pl
from jax.experimental.pallas import tpu as pltpu
from jax.experimental.pallas import tpu_sc as plsc

NUM_FIELDS = 26
VOCAB = 100000
EMB_DIM = 32
BATCH = 16384

_INFO = plsc.get_sparse_core_info()
_NC, _NS, _L = _INFO.num_cores, _INFO.num_subcores, _INFO.num_lanes
_NW = _NC * _NS
_BPW = BATCH // _NW
_CHUNK = 512
_NCHUNK = _BPW // _CHUNK


def _sc_embedding(xt, flat_tables):
    mesh = plsc.VectorSubcoreMesh(core_axis_name="c", subcore_axis_name="s")

    @functools.partial(
        pl.kernel,
        mesh=mesh,
        out_type=jax.ShapeDtypeStruct((BATCH, NUM_FIELDS, EMB_DIM),
                                      jnp.float32),
        scratch_types=[
            pltpu.VMEM((NUM_FIELDS, _BPW), jnp.int32),
            pltpu.VMEM((_NCHUNK, _CHUNK), jnp.int32),
            pltpu.VMEM((_NCHUNK, _CHUNK), jnp.int32),
            pltpu.VMEM((_NCHUNK, _CHUNK), jnp.int32),
            pltpu.VMEM((_NCHUNK, _CHUNK, EMB_DIM), jnp.float32),
            pltpu.SemaphoreType.DMA,
            pltpu.SemaphoreType.DMA,
        ],
        compiler_params=pltpu.CompilerParams(use_tc_tiling_on_sc=False),
    )
    def k(xt_hbm, tab_hbm, out_hbm, idx_v, goff_v, woff_v, ramp_v, rows_v,
          gsem, wsem):
        wid = lax.axis_index("s") * _NC + lax.axis_index("c")
        base = wid * _BPW

        pltpu.sync_copy(xt_hbm.at[:, pl.ds(base, _BPW)], idx_v)

        for c in range(_NCHUNK):
            for j in range(_CHUNK // _L):
                sl = pl.ds(j * _L, _L)
                ramp_v[c, sl] = (
                    lax.iota(jnp.int32, _L) + (c * _CHUNK + j * _L)
                ) * NUM_FIELDS

        def fbody(f, carry):
            foff = f * VOCAB
            for c in range(_NCHUNK):
                for j in range(_CHUNK // _L):
                    sl = pl.ds(j * _L, _L)
                    raw = idx_v[f, pl.ds(c * _CHUNK + j * _L, _L)]
                    goff_v[c, sl] = raw + foff
            for c in range(_NCHUNK):
                pltpu.async_copy(tab_hbm.at[goff_v.at[c]], rows_v.at[c],
                                 gsem).wait()
                pltpu.async_copy(
                    rows_v.at[c],
                    out_hbm.at[pl.ds(base + c * _CHUNK, _CHUNK), f],
                    wsem).wait()
            return carry

        lax.fori_loop(0, NUM_FIELDS, fbody, 0)

    return k(xt, flat_tables)


def kernel(x_cat, tables):
    xt = x_cat.T.astype(jnp.int32)
    flat = tables.reshape(NUM_FIELDS * VOCAB, EMB_DIM)
    out = _sc_embedding(xt, flat)
    return out.reshape(BATCH, NUM_FIELDS * EMB_DIM)


# speedup vs baseline: 1.1141x; 1.1141x over previous
import functools

import jax
import jax.numpy as jnp
from jax import lax
from jax.experimental import pallas as pl
from jax.experimental.pallas import tpu as pltpu
from jax.experimental.pallas import tpu_sc as plsc

NUM_FIELDS = 26
VOCAB = 100000
EMB_DIM = 32
BATCH = 16384

_INFO = plsc.get_sparse_core_info()
_NC, _NS, _L = _INFO.num_cores, _INFO.num_subcores, _INFO.num_lanes
_NW = _NC * _NS
_BPW = BATCH // _NW
_NPAIR = NUM_FIELDS // 2


def _sc_embedding(xt, flat_tables):
    mesh = plsc.VectorSubcoreMesh(core_axis_name="c", subcore_axis_name="s")

    @functools.partial(
        pl.kernel,
        mesh=mesh,
        out_type=jax.ShapeDtypeStruct((BATCH * NUM_FIELDS, EMB_DIM),
                                      jnp.float32),
        scratch_types=[
            pltpu.VMEM((NUM_FIELDS, _BPW), jnp.int32),
            pltpu.VMEM((2, _BPW), jnp.int32),
            pltpu.VMEM((2, _BPW), jnp.int32),
            pltpu.VMEM((_BPW,), jnp.int32),
            pltpu.VMEM((2, _BPW, EMB_DIM), jnp.float32),
            pltpu.SemaphoreType.DMA,
            pltpu.SemaphoreType.DMA,
            pltpu.SemaphoreType.DMA,
            pltpu.SemaphoreType.DMA,
        ],
        compiler_params=pltpu.CompilerParams(use_tc_tiling_on_sc=False),
    )
    def k(xt_hbm, tab_hbm, out_hbm, idx_v, goff_v, woff_v, ramp_v, rows_v,
          gsem0, gsem1, wsem0, wsem1):
        wid = lax.axis_index("s") * _NC + lax.axis_index("c")
        base = wid * _BPW

        pltpu.sync_copy(xt_hbm.at[:, pl.ds(base, _BPW)], idx_v)

        for j in range(_BPW // _L):
            ramp_v[pl.ds(j * _L, _L)] = (
                lax.iota(jnp.int32, _L) + j * _L) * NUM_FIELDS

        def offsets(slot, f):
            foff = f * VOCAB
            wbase = base * NUM_FIELDS + f
            for j in range(_BPW // _L):
                sl = pl.ds(j * _L, _L)
                goff_v[slot, sl] = idx_v[f, sl] + foff
                woff_v[slot, sl] = ramp_v[sl] + wbase

        def gather(slot, sem):
            return pltpu.async_copy(tab_hbm.at[goff_v.at[slot]],
                                    rows_v.at[slot], sem)

        def scatter(slot, sem):
            return pltpu.async_copy(rows_v.at[slot],
                                    out_hbm.at[woff_v.at[slot]], sem)

        def drain_scatter(slot, sem):
            pltpu.make_async_copy(tab_hbm.at[pl.ds(0, _BPW)],
                                  rows_v.at[slot], sem).wait()

        def pbody(g, carry):
            f0 = 2 * g
            f1 = 2 * g + 1

            @pl.when(g > 0)
            def _():
                drain_scatter(0, wsem0)

            offsets(0, f0)
            g0 = gather(0, gsem0)

            @pl.when(g > 0)
            def _():
                drain_scatter(1, wsem1)

            offsets(1, f1)
            g1 = gather(1, gsem1)

            g0.wait()
            scatter(0, wsem0)
            g1.wait()
            scatter(1, wsem1)
            return carry

        lax.fori_loop(0, _NPAIR, pbody, 0)
        drain_scatter(0, wsem0)
        drain_scatter(1, wsem1)

    return k(xt, flat_tables)


def kernel(x_cat, tables):
    xt = x_cat.T.astype(jnp.int32)
    flat = tables.reshape(NUM_FIELDS * VOCAB, EMB_DIM)
    out = _sc_embedding(xt, flat)
    return out.reshape(BATCH, NUM_FIELDS * EMB_DIM)


# speedup vs baseline: 1.1142x; 1.0001x over previous
import functools

import jax
import jax.numpy as jnp
from jax import lax
from jax.experimental import pallas as pl
from jax.experimental.pallas import tpu as pltpu
from jax.experimental.pallas import tpu_sc as plsc

NUM_FIELDS = 26
VOCAB = 100000
EMB_DIM = 32
BATCH = 16384

_INFO = plsc.get_sparse_core_info()
_NC, _NS, _L = _INFO.num_cores, _INFO.num_subcores, _INFO.num_lanes
_NW = _NC * _NS
_BPW = BATCH // _NW


def _sc_embedding(xt, flat_tables):
    mesh = plsc.VectorSubcoreMesh(core_axis_name="c", subcore_axis_name="s")

    @functools.partial(
        pl.kernel,
        mesh=mesh,
        out_type=jax.ShapeDtypeStruct((BATCH * NUM_FIELDS, EMB_DIM),
                                      jnp.float32),
        scratch_types=[
            pltpu.VMEM((NUM_FIELDS, _BPW), jnp.int32),
            pltpu.VMEM((2, _BPW), jnp.int32),
            pltpu.VMEM((2, _BPW, EMB_DIM), jnp.float32),
            pltpu.SemaphoreType.DMA,
            pltpu.SemaphoreType.DMA,
            pltpu.SemaphoreType.DMA,
            pltpu.SemaphoreType.DMA,
        ],
        compiler_params=pltpu.CompilerParams(use_tc_tiling_on_sc=False),
    )
    def k(xt_hbm, tab_hbm, out_hbm, idx_v, goff_v, rows_v,
          gsem0, gsem1, wsem0, wsem1):
        wid = lax.axis_index("s") * _NC + lax.axis_index("c")
        base = wid * _BPW

        pltpu.sync_copy(xt_hbm.at[:, pl.ds(base, _BPW)], idx_v)

        def offsets(slot, f):
            foff = f * VOCAB
            for j in range(_BPW // _L):
                sl = pl.ds(j * _L, _L)
                goff_v[slot, sl] = idx_v[f, sl] + foff

        def gather(slot, sem):
            return pltpu.async_copy(tab_hbm.at[goff_v.at[slot]],
                                    rows_v.at[slot], sem)

        def lin_write(slot, f, sem):
            return pltpu.async_copy(
                rows_v.at[slot],
                out_hbm.at[pl.ds(base * NUM_FIELDS + f * _BPW, _BPW)], sem)

        def drain(slot, sem):
            pltpu.make_async_copy(tab_hbm.at[pl.ds(0, _BPW)],
                                  rows_v.at[slot], sem).wait()

        def pbody(g, carry):
            f0 = 2 * g
            f1 = 2 * g + 1

            @pl.when(g > 0)
            def _():
                drain(0, wsem0)

            offsets(0, f0)
            g0 = gather(0, gsem0)

            @pl.when(g > 0)
            def _():
                drain(1, wsem1)

            offsets(1, f1)
            g1 = gather(1, gsem1)

            g0.wait()
            lin_write(0, f0, wsem0)
            g1.wait()
            lin_write(1, f1, wsem1)
            return carry

        lax.fori_loop(0, NUM_FIELDS // 2, pbody, 0)
        drain(0, wsem0)
        drain(1, wsem1)

    return k(xt, flat_tables)


def kernel(x_cat, tables):
    xt = x_cat.T.astype(jnp.int32)
    flat = tables.reshape(NUM_FIELDS * VOCAB, EMB_DIM)
    out = _sc_embedding(xt, flat)
    return out.reshape(BATCH, NUM_FIELDS * EMB_DIM)


# speedup vs baseline: 1.1150x; 1.0007x over previous
import functools

import jax
import jax.numpy as jnp
from jax import lax
from jax.experimental import pallas as pl
from jax.experimental.pallas import tpu as pltpu
from jax.experimental.pallas import tpu_sc as plsc

NUM_FIELDS = 26
VOCAB = 100000
EMB_DIM = 32
BATCH = 16384

_INFO = plsc.get_sparse_core_info()
_NC, _NS, _L = _INFO.num_cores, _INFO.num_subcores, _INFO.num_lanes
_NW = _NC * _NS
_BPW = BATCH // _NW


def _sc_embedding(xt, tables):
    mesh = plsc.VectorSubcoreMesh(core_axis_name="c", subcore_axis_name="s")

    @functools.partial(
        pl.kernel,
        mesh=mesh,
        out_type=jax.ShapeDtypeStruct((BATCH * NUM_FIELDS, EMB_DIM),
                                      jnp.float32),
        scratch_types=[
            pltpu.VMEM((NUM_FIELDS, _BPW), jnp.int32),
            pltpu.VMEM((2, _BPW), jnp.int32),
            pltpu.VMEM((_BPW,), jnp.int32),
            pltpu.VMEM((2, _BPW, EMB_DIM), jnp.float32),
            pltpu.SemaphoreType.DMA,
            pltpu.SemaphoreType.DMA,
            pltpu.SemaphoreType.DMA,
            pltpu.SemaphoreType.DMA,
        ],
        compiler_params=pltpu.CompilerParams(use_tc_tiling_on_sc=False),
    )
    def k(xt_hbm, tab_hbm, out_hbm, idx_v, woff_v, ramp_v, rows_v,
          gsem0, gsem1, wsem0, wsem1):
        wid = lax.axis_index("s") * _NC + lax.axis_index("c")
        base = wid * _BPW

        pltpu.sync_copy(xt_hbm.at[:, pl.ds(base, _BPW)], idx_v)

        for j in range(_BPW // _L):
            ramp_v[pl.ds(j * _L, _L)] = (
                lax.iota(jnp.int32, _L) + j * _L) * NUM_FIELDS

        def offsets(slot, f):
            wbase = base * NUM_FIELDS + f
            for j in range(_BPW // _L):
                sl = pl.ds(j * _L, _L)
                woff_v[slot, sl] = ramp_v[sl] + wbase

        def gather(slot, f, sem):
            return pltpu.async_copy(tab_hbm.at[f].at[idx_v.at[f]],
                                    rows_v.at[slot], sem)

        def scatter(slot, sem):
            return pltpu.async_copy(rows_v.at[slot],
                                    out_hbm.at[woff_v.at[slot]], sem)

        def drain_scatter(slot, sem):
            pltpu.make_async_copy(tab_hbm.at[0].at[pl.ds(0, _BPW)],
                                  rows_v.at[slot], sem).wait()

        def pbody(g, carry):
            f0 = 2 * g
            f1 = 2 * g + 1

            @pl.when(g > 0)
            def _():
                drain_scatter(0, wsem0)

            offsets(0, f0)
            g0 = gather(0, f0, gsem0)

            @pl.when(g > 0)
            def _():
                drain_scatter(1, wsem1)

            offsets(1, f1)
            g1 = gather(1, f1, gsem1)

            g0.wait()
            scatter(0, wsem0)
            g1.wait()
            scatter(1, wsem1)
            return carry

        lax.fori_loop(0, NUM_FIELDS // 2, pbody, 0)
        drain_scatter(0, wsem0)
        drain_scatter(1, wsem1)

    return k(xt, tables)


def kernel(x_cat, tables):
    xt = x_cat.T.astype(jnp.int32)
    out = _sc_embedding(xt, tables)
    return out.reshape(BATCH, NUM_FIELDS * EMB_DIM)


# speedup vs baseline: 1.6613x; 1.4900x over previous
import functools

import jax
import jax.numpy as jnp
from jax import lax
from jax.experimental import pallas as pl
from jax.experimental.pallas import tpu as pltpu
from jax.experimental.pallas import tpu_sc as plsc

NUM_FIELDS = 26
VOCAB = 100000
EMB_DIM = 32
BATCH = 16384

_INFO = plsc.get_sparse_core_info()
_NC, _NS, _L = _INFO.num_cores, _INFO.num_subcores, _INFO.num_lanes
_NW = _NC * _NS
_HALF = BATCH // 2


def _sc_embedding_t(xt, tabt):
    mesh = plsc.VectorSubcoreMesh(core_axis_name="c", subcore_axis_name="s")

    @functools.partial(
        pl.kernel,
        mesh=mesh,
        out_type=jax.ShapeDtypeStruct((NUM_FIELDS * EMB_DIM, BATCH),
                                      jnp.float32),
        scratch_types=[
            pltpu.VMEM((VOCAB,), jnp.float32),
            pltpu.VMEM((_HALF,), jnp.int32),
            pltpu.VMEM((2, _HALF), jnp.float32),
            pltpu.SemaphoreType.DMA,
            pltpu.SemaphoreType.DMA,
        ],
        compiler_params=pltpu.CompilerParams(use_tc_tiling_on_sc=False,
                                             needs_layout_passes=False),
    )
    def k(xt_hbm, tab_hbm, out_hbm, vocab_v, idx_v, outr_v, gsem, wsem):
        d = lax.axis_index("s") * _NC + lax.axis_index("c")

        def drain_write(slot):
            pltpu.make_async_copy(xt_hbm.at[0, pl.ds(0, _HALF)],
                                  outr_v.at[slot], wsem).wait()

        def fbody(f, carry):
            row = f * EMB_DIM + d
            pltpu.sync_copy(tab_hbm.at[row], vocab_v)
            for h in range(2):
                pltpu.sync_copy(xt_hbm.at[f, pl.ds(h * _HALF, _HALF)], idx_v)

                @pl.when(f > 0)
                def _():
                    drain_write(h)

                def gbody(j, c):
                    sl = pl.ds(j * _L, _L)
                    iv = idx_v[sl]
                    outr_v[h, sl] = plsc.load_gather(vocab_v, [iv])
                    return c

                lax.fori_loop(0, _HALF // _L, gbody, 0)
                pltpu.async_copy(outr_v.at[h],
                                 out_hbm.at[row, pl.ds(h * _HALF, _HALF)],
                                 wsem)
            return carry

        lax.fori_loop(0, NUM_FIELDS, fbody, 0)
        drain_write(0)
        drain_write(1)

    return k(xt, tabt)


def kernel(x_cat, tables):
    xt = x_cat.T.astype(jnp.int32)
    tabt = tables.transpose(0, 2, 1).reshape(
        NUM_FIELDS * EMB_DIM, VOCAB)
    out_t = _sc_embedding_t(xt, tabt)
    return out_t.T.reshape(BATCH, NUM_FIELDS * EMB_DIM)
